# Initial kernel scaffold; baseline (speedup 1.0000x reference)
import jax
import jax.numpy as jnp
from jax.experimental import pallas as pl


def kernel(x, edge_index, batch, W_emb, b_emb, Wl1, bl1, Wr1, Wl2, bl2, Wr2, Wl3, bl3, Wr3, W1, b1, W2, b2):
    raise NotImplementedError("write your pallas kernel here")


# speedup vs baseline: 1.8551x; 1.8551x over previous
import functools

import jax
import jax.numpy as jnp
from jax import lax
from jax.experimental import pallas as pl
from jax.experimental.pallas import tpu as pltpu
from jax.experimental.pallas import tpu_sc as plsc

N = 100000
N_PAD = 102400
E = 1600000
E_PAD = 1638400
H = 128
D_IN = 16
G = 64
G_PAD = 128
NC = 2
NS = 16
NW = NC * NS
CH = 16
NCH = H // CH
EB = 1024
NEB = E_PAD // NW // EB
ROWS_PER_TILE = N_PAD // NS

_mesh = plsc.VectorSubcoreMesh(core_axis_name="c", subcore_axis_name="s")


@functools.partial(
    pl.kernel,
    out_type=jax.ShapeDtypeStruct((NC, NCH + 1, N_PAD, CH), jnp.float32),
    mesh=_mesh,
    scratch_types=[
        pltpu.VMEM((EB,), jnp.int32),
        pltpu.VMEM((EB,), jnp.int32),
        pltpu.VMEM((EB // 128, 128), jnp.int32),
        pltpu.VMEM((EB, CH), jnp.float32),
        pltpu.VMEM((400, CH), jnp.float32),
        pltpu.VMEM((128, CH), jnp.float32),
        pltpu.VMEM_SHARED((N_PAD, CH), jnp.float32),
        pltpu.SemaphoreType.DMA,
    ],
    compiler_params=pltpu.CompilerParams(use_tc_tiling_on_sc=False),
)
def _sc_agg(h_flat, src, dst2d, zeros_hbm, ones_hbm, agg_out,
            src_v, idx_v, dst_v, rows_v, zbuf, ones_v, acc, sem):
    cid = lax.axis_index("c")
    sid = lax.axis_index("s")
    wid = sid * NC + cid

    pltpu.sync_copy(zeros_hbm, zbuf)
    pltpu.sync_copy(ones_hbm, ones_v)

    def zero_acc():
        def zrow(r, _):
            pltpu.sync_copy(
                zbuf,
                acc.at[pl.ds(pl.multiple_of(sid * ROWS_PER_TILE + r * 400, 400),
                             400), :])
            return 0
        lax.fori_loop(0, ROWS_PER_TILE // 400, zrow, 0)

    def writeout(k):
        for r in range(ROWS_PER_TILE // 3200):
            row0 = sid * ROWS_PER_TILE + r * 3200
            pltpu.sync_copy(acc.at[pl.ds(row0, 3200), :],
                            agg_out.at[cid, k, pl.ds(row0, 3200), :])

    def load_dst(off):
        pltpu.sync_copy(
            dst2d.at[pl.ds(pl.multiple_of(off // 128, EB // 128), EB // 128), :],
            dst_v)

    def chunk_body(k, _):
        zero_acc()
        plsc.subcore_barrier()

        def edge_block(b, _):
            off = pl.multiple_of(wid * (E_PAD // NW) + b * EB, EB)
            pltpu.sync_copy(src.at[pl.ds(off, EB)], src_v)
            load_dst(off)
            for i in range(EB // 16):
                s = src_v[pl.ds(i * 16, 16)]
                idx_v[pl.ds(i * 16, 16)] = s * NCH + k
            descs = []
            for j in range(EB // 128):
                descs.append(pltpu.async_copy(
                    h_flat.at[idx_v.at[pl.ds(j * 128, 128)]],
                    rows_v.at[pl.ds(j * 128, 128), :], sem))
            for d in descs:
                d.wait()
            for j in range(EB // 128):
                pltpu.sync_copy(rows_v.at[pl.ds(j * 128, 128), :],
                                acc.at[dst_v.at[j]], add=True)
            return 0

        lax.fori_loop(0, NEB, edge_block, 0)
        plsc.subcore_barrier()
        writeout(k)
        plsc.subcore_barrier()
        return 0

    lax.fori_loop(0, NCH, chunk_body, 0)

    zero_acc()
    plsc.subcore_barrier()

    def deg_block(b, _):
        off = pl.multiple_of(wid * (E_PAD // NW) + b * EB, EB)
        load_dst(off)
        for j in range(EB // 128):
            pltpu.sync_copy(ones_v, acc.at[dst_v.at[j]], add=True)
        return 0

    lax.fori_loop(0, NEB, deg_block, 0)
    plsc.subcore_barrier()
    writeout(NCH)


@functools.partial(
    pl.kernel,
    out_type=(jax.ShapeDtypeStruct((NC, G_PAD, H), jnp.float32),
              jax.ShapeDtypeStruct((NC, G_PAD, CH), jnp.float32)),
    mesh=_mesh,
    scratch_types=[
        pltpu.VMEM((128, H), jnp.float32),
        pltpu.VMEM((N_PAD // NW // 128, 128), jnp.int32),
        pltpu.VMEM((128, CH), jnp.float32),
        pltpu.VMEM_SHARED((G_PAD, H), jnp.float32),
        pltpu.VMEM_SHARED((G_PAD, CH), jnp.float32),
    ],
    compiler_params=pltpu.CompilerParams(use_tc_tiling_on_sc=False),
)
def _sc_pool(h3, batch3d, zeros_s, zeros_c, ones_hbm, sums_out, cnts_out,
             hbuf, bidx, ones_v, acc_s, acc_c):
    cid = lax.axis_index("c")
    sid = lax.axis_index("s")
    wid = sid * NC + cid

    pltpu.sync_copy(ones_hbm, ones_v)

    @pl.when(sid == 0)
    def _():
        pltpu.sync_copy(zeros_s, acc_s)
        pltpu.sync_copy(zeros_c, acc_c)
    plsc.subcore_barrier()

    pltpu.sync_copy(batch3d.at[wid], bidx)

    def block(b, _):
        row0 = pl.multiple_of(wid * (N_PAD // NW) + b * 128, 128)
        pltpu.sync_copy(h3.at[pl.ds(row0, 128), :], hbuf)
        pltpu.sync_copy(hbuf, acc_s.at[bidx.at[b]], add=True)
        pltpu.sync_copy(ones_v, acc_c.at[bidx.at[b]], add=True)
        return 0

    lax.fori_loop(0, N_PAD // NW // 128, block, 0)
    plsc.subcore_barrier()

    @pl.when(sid == 0)
    def _():
        pltpu.sync_copy(acc_s, sums_out.at[cid])
        pltpu.sync_copy(acc_c, cnts_out.at[cid])


_RB = 512
_NRB = N_PAD // _RB


def _emb_body(x_ref, w_ref, b_ref, o_ref):
    o_ref[...] = jnp.dot(x_ref[...], w_ref[...],
                         preferred_element_type=jnp.float32,
                         precision=lax.Precision.HIGHEST) + b_ref[...]


def _tc_emb(x_pad, w, b):
    return pl.pallas_call(
        _emb_body,
        grid=(_NRB,),
        in_specs=[
            pl.BlockSpec((_RB, D_IN), lambda i: (i, 0)),
            pl.BlockSpec((D_IN, H), lambda i: (0, 0)),
            pl.BlockSpec((1, H), lambda i: (0, 0)),
        ],
        out_specs=pl.BlockSpec((_RB, H), lambda i: (i, 0)),
        out_shape=jax.ShapeDtypeStruct((N_PAD, H), jnp.float32),
    )(x_pad, w, b)


def _layer_body(aggA_ref, aggB_ref, degA_ref, degB_ref, h_ref,
                wl_ref, bl_ref, wr_ref, o_ref):
    d = degA_ref[0, 0][:, 0:1] + degB_ref[0, 0][:, 0:1]
    inv = 1.0 / jnp.maximum(d, 1.0)
    agg = jnp.concatenate(
        [aggA_ref[0, j] + aggB_ref[0, j] for j in range(NCH)], axis=-1)
    mean = agg * inv
    z = (jnp.dot(mean, wl_ref[...], preferred_element_type=jnp.float32,
                 precision=lax.Precision.HIGHEST)
         + bl_ref[...]
         + jnp.dot(h_ref[...], wr_ref[...], preferred_element_type=jnp.float32,
                   precision=lax.Precision.HIGHEST))
    o_ref[...] = jnp.maximum(z, 0.0)


def _tc_layer(agg_parts, h, wl, bl, wr):
    return pl.pallas_call(
        _layer_body,
        grid=(_NRB,),
        in_specs=[
            pl.BlockSpec((1, NCH, _RB, CH), lambda i: (0, 0, i, 0)),
            pl.BlockSpec((1, NCH, _RB, CH), lambda i: (1, 0, i, 0)),
            pl.BlockSpec((1, 1, _RB, CH), lambda i: (0, NCH, i, 0)),
            pl.BlockSpec((1, 1, _RB, CH), lambda i: (1, NCH, i, 0)),
            pl.BlockSpec((_RB, H), lambda i: (i, 0)),
            pl.BlockSpec((H, H), lambda i: (0, 0)),
            pl.BlockSpec((1, H), lambda i: (0, 0)),
            pl.BlockSpec((H, H), lambda i: (0, 0)),
        ],
        out_specs=pl.BlockSpec((_RB, H), lambda i: (i, 0)),
        out_shape=jax.ShapeDtypeStruct((N_PAD, H), jnp.float32),
    )(agg_parts, agg_parts, agg_parts, agg_parts, h, wl, bl, wr)


def _head_body(s_ref, c_ref, w1_ref, b1_ref, w2_ref, o_ref):
    s = s_ref[0] + s_ref[1]
    c = c_ref[0][:, 0:1] + c_ref[1][:, 0:1]
    pooled = s * (1.0 / jnp.maximum(c, 1.0))
    z = jnp.maximum(
        jnp.dot(pooled, w1_ref[...], preferred_element_type=jnp.float32,
                precision=lax.Precision.HIGHEST) + b1_ref[...], 0.0)
    o_ref[...] = jnp.dot(z, w2_ref[...], preferred_element_type=jnp.float32,
                         precision=lax.Precision.HIGHEST)


def _tc_head(sums, cnts, w1, b1, w2p):
    return pl.pallas_call(
        _head_body,
        grid=(1,),
        in_specs=[
            pl.BlockSpec((NC, G_PAD, H), lambda i: (0, 0, 0)),
            pl.BlockSpec((NC, G_PAD, CH), lambda i: (0, 0, 0)),
            pl.BlockSpec((H, H), lambda i: (0, 0)),
            pl.BlockSpec((1, H), lambda i: (0, 0)),
            pl.BlockSpec((H, H), lambda i: (0, 0)),
        ],
        out_specs=pl.BlockSpec((G_PAD, H), lambda i: (0, 0)),
        out_shape=jax.ShapeDtypeStruct((G_PAD, H), jnp.float32),
    )(sums, cnts, w1, b1, w2p)


@jax.jit
def _impl(x, edge_index, batch, W_emb, b_emb, Wl1, bl1, Wr1, Wl2, bl2, Wr2,
          Wl3, bl3, Wr3, W1, b1, W2, b2):
    src = edge_index[0].astype(jnp.int32)
    dst = edge_index[1].astype(jnp.int32)
    src_pad = jnp.concatenate([src, jnp.zeros((E_PAD - E,), jnp.int32)])
    dst_pad = jnp.concatenate([dst, jnp.full((E_PAD - E,), N, jnp.int32)])
    dst2d = dst_pad.reshape(E_PAD // 128, 128)
    batch3d = jnp.concatenate(
        [batch.astype(jnp.int32), jnp.full((N_PAD - N,), G, jnp.int32)]
    ).reshape(NW, N_PAD // NW // 128, 128)
    x_pad = jnp.pad(x, ((0, N_PAD - N), (0, 0)))

    zeros400 = jnp.zeros((400, CH), jnp.float32)
    ones128 = jnp.ones((128, CH), jnp.float32)
    zeros_s = jnp.zeros((G_PAD, H), jnp.float32)
    zeros_c = jnp.zeros((G_PAD, CH), jnp.float32)

    h0 = _tc_emb(x_pad, W_emb, b_emb.reshape(1, H))

    wls = jnp.stack([Wl1, Wl2, Wl3])
    bls = jnp.stack([bl1.reshape(1, H), bl2.reshape(1, H), bl3.reshape(1, H)])
    wrs = jnp.stack([Wr1, Wr2, Wr3])

    def layer_step(h, ws):
        wl, bl, wr = ws
        agg_parts = _sc_agg(h.reshape(N_PAD * NCH, CH), src_pad, dst2d,
                            zeros400, ones128)
        return _tc_layer(agg_parts, h, wl, bl, wr), 0.0

    h, _ = lax.scan(layer_step, h0, (wls, bls, wrs))

    sums, cnts = _sc_pool(h, batch3d, zeros_s, zeros_c, ones128)
    w2p = jnp.pad(W2, ((0, 0), (0, H - 1)))
    out = _tc_head(sums, cnts, W1, b1.reshape(1, H), w2p)
    return out[:G, 0] + b2[0]


def kernel(x, edge_index, batch, W_emb, b_emb, Wl1, bl1, Wr1, Wl2, bl2, Wr2,
           Wl3, bl3, Wr3, W1, b1, W2, b2):
    return _impl(x, edge_index, batch, W_emb, b_emb, Wl1, bl1, Wr1,
                 Wl2, bl2, Wr2, Wl3, bl3, Wr3, W1, b1, W2, b2)
